# baseline (speedup 1.0000x reference)
import jax
import jax.numpy as jnp
from jax.experimental import pallas as pl


H, W = 384, 384
PLANES_PER_BLOCK = 24


def _body(x_ref, m_ref, o_ref):
    o_ref[...] = x_ref[...] * m_ref[None]


def kernel(x, mask):
    B, C = x.shape[0], x.shape[1]
    n_planes = B * C
    xf = x.reshape(n_planes, H, W)
    out = pl.pallas_call(
        _body,
        grid=(n_planes // PLANES_PER_BLOCK,),
        in_specs=[
            pl.BlockSpec((PLANES_PER_BLOCK, H, W), lambda i: (i, 0, 0)),
            pl.BlockSpec((H, W), lambda i: (0, 0)),
        ],
        out_specs=pl.BlockSpec((PLANES_PER_BLOCK, H, W), lambda i: (i, 0, 0)),
        out_shape=jax.ShapeDtypeStruct((n_planes, H, W), x.dtype),
    )(xf, mask)
    return out.reshape(B, C, H, W)


# speedup vs baseline: 1.0032x; 1.0032x over previous
import jax
import jax.numpy as jnp
from jax.experimental import pallas as pl
from jax.experimental.pallas import tpu as pltpu


H, W = 384, 384
PLANES_PER_BLOCK = 26


def _body(x_ref, m_ref, o_ref):
    o_ref[...] = x_ref[...] * m_ref[None]


def kernel(x, mask):
    B, C = x.shape[0], x.shape[1]
    n_planes = B * C
    xf = x.reshape(n_planes, H, W)
    out = pl.pallas_call(
        _body,
        grid=(-(-n_planes // PLANES_PER_BLOCK),),
        in_specs=[
            pl.BlockSpec((PLANES_PER_BLOCK, H, W), lambda i: (i, 0, 0)),
            pl.BlockSpec((H, W), lambda i: (0, 0)),
        ],
        out_specs=pl.BlockSpec((PLANES_PER_BLOCK, H, W), lambda i: (i, 0, 0)),
        out_shape=jax.ShapeDtypeStruct((n_planes, H, W), x.dtype),
        compiler_params=pltpu.CompilerParams(vmem_limit_bytes=100 * 1024 * 1024),
    )(xf, mask)
    return out.reshape(B, C, H, W)
